# baseline (speedup 1.0000x reference)
import functools

import jax
import jax.numpy as jnp
from jax import lax
from jax.experimental import pallas as pl
from jax.experimental.pallas import tpu as pltpu
from jax.experimental.pallas import tpu_sc as plsc

_D = 64
_B = 4096
_NC = 2
_NS = 16
_NW = _NC * _NS
_BPW = _B // _NW

_mesh = plsc.VectorSubcoreMesh(core_axis_name="c", subcore_axis_name="s")


@functools.partial(
    pl.kernel,
    out_type=(
        jax.ShapeDtypeStruct((_B, _D), jnp.float32),
        jax.ShapeDtypeStruct((_B, _D), jnp.float32),
    ),
    mesh=_mesh,
    scratch_types=[
        pltpu.VMEM((_BPW,), jnp.int32),
        pltpu.VMEM((_BPW,), jnp.int32),
        pltpu.VMEM((_BPW, _D), jnp.float32),
        pltpu.VMEM((_BPW, _D), jnp.float32),
        pltpu.SemaphoreType.DMA,
        pltpu.SemaphoreType.DMA,
    ],
    compiler_params=pltpu.CompilerParams(use_tc_tiling_on_sc=False),
)
def _sc_gather(user_idx_hbm, item_idx_hbm, user_emb_hbm, item_emb_hbm,
               wu_out, hi_out, uidx_v, iidx_v, urows_v, irows_v, usem, isem):
    wid = lax.axis_index("s") * _NC + lax.axis_index("c")
    base = wid * _BPW
    pltpu.sync_copy(user_idx_hbm.at[pl.ds(base, _BPW)], uidx_v)
    pltpu.sync_copy(item_idx_hbm.at[pl.ds(base, _BPW)], iidx_v)
    ucopy = pltpu.async_copy(user_emb_hbm.at[uidx_v], urows_v, usem)
    icopy = pltpu.async_copy(item_emb_hbm.at[iidx_v], irows_v, isem)
    ucopy.wait()
    pltpu.sync_copy(urows_v, wu_out.at[pl.ds(base, _BPW)])
    icopy.wait()
    pltpu.sync_copy(irows_v, hi_out.at[pl.ds(base, _BPW)])


_BN = 512


def _mm_body(wu_ref, hi_ref, out_ref):
    out_ref[...] = lax.dot_general(
        wu_ref[...], hi_ref[...],
        (((1,), (1,)), ((), ())),
        preferred_element_type=jnp.float32,
    )


_matmul = pl.pallas_call(
    _mm_body,
    grid=(_B // _BN,),
    in_specs=[
        pl.BlockSpec((_B, _D), lambda j: (0, 0)),
        pl.BlockSpec((_BN, _D), lambda j: (j, 0)),
    ],
    out_specs=pl.BlockSpec((_B, _BN), lambda j: (0, j)),
    out_shape=jax.ShapeDtypeStruct((_B, _B), jnp.float32),
)


@jax.jit
def kernel(user_tensor, item_tensor, user_embedding, item_embedding):
    w_u, h_i = _sc_gather(
        user_tensor.astype(jnp.int32), item_tensor.astype(jnp.int32),
        user_embedding, item_embedding)
    return _matmul(w_u, h_i)


# speedup vs baseline: 1.0045x; 1.0045x over previous
import functools

import jax
import jax.numpy as jnp
from jax import lax
from jax.experimental import pallas as pl
from jax.experimental.pallas import tpu as pltpu
from jax.experimental.pallas import tpu_sc as plsc

_D = 64
_B = 4096
_NC = 2
_NS = 16
_NW = _NC * _NS
_BPW = _B // _NW

_mesh = plsc.VectorSubcoreMesh(core_axis_name="c", subcore_axis_name="s")


@functools.partial(
    pl.kernel,
    out_type=(
        jax.ShapeDtypeStruct((_B, 2 * _D), jnp.float32),
        jax.ShapeDtypeStruct((_B, 2 * _D), jnp.float32),
    ),
    mesh=_mesh,
    scratch_types=[
        pltpu.VMEM((_BPW,), jnp.int32),
        pltpu.VMEM((_BPW,), jnp.int32),
        pltpu.VMEM((_BPW, 2 * _D), jnp.float32),
        pltpu.VMEM((_BPW, 2 * _D), jnp.float32),
        pltpu.SemaphoreType.DMA,
        pltpu.SemaphoreType.DMA,
    ],
)
def _sc_gather(user_idx_hbm, item_idx_hbm, user_emb_hbm, item_emb_hbm,
               wu_out, hi_out, uidx_v, iidx_v, urows_v, irows_v, usem, isem):
    wid = lax.axis_index("s") * _NC + lax.axis_index("c")
    base = wid * _BPW
    pltpu.sync_copy(user_idx_hbm.at[pl.ds(base, _BPW)], uidx_v)
    pltpu.sync_copy(item_idx_hbm.at[pl.ds(base, _BPW)], iidx_v)
    ucopy = pltpu.async_copy(user_emb_hbm.at[uidx_v], urows_v, usem)
    icopy = pltpu.async_copy(item_emb_hbm.at[iidx_v], irows_v, isem)
    ucopy.wait()
    pltpu.sync_copy(urows_v, wu_out.at[pl.ds(base, _BPW)])
    icopy.wait()
    pltpu.sync_copy(irows_v, hi_out.at[pl.ds(base, _BPW)])


_BN = 512


def _mm_body(wu2_ref, pu_ref, hi2_ref, pi_ref, out_ref):
    pu = pu_ref[...]
    wu = wu2_ref[:, :_D] * (1.0 - pu) + wu2_ref[:, _D:] * pu
    pi = pi_ref[...]
    hi = hi2_ref[:, :_D] * (1.0 - pi) + hi2_ref[:, _D:] * pi
    out_ref[...] = lax.dot_general(
        wu, hi,
        (((1,), (1,)), ((), ())),
        preferred_element_type=jnp.float32,
    )


_matmul = pl.pallas_call(
    _mm_body,
    grid=(_B // _BN,),
    in_specs=[
        pl.BlockSpec((_B, 2 * _D), lambda j: (0, 0)),
        pl.BlockSpec((_B, _D), lambda j: (0, 0)),
        pl.BlockSpec((_BN, 2 * _D), lambda j: (j, 0)),
        pl.BlockSpec((_BN, _D), lambda j: (j, 0)),
    ],
    out_specs=pl.BlockSpec((_B, _BN), lambda j: (0, j)),
    out_shape=jax.ShapeDtypeStruct((_B, _B), jnp.float32),
)


@jax.jit
def kernel(user_tensor, item_tensor, user_embedding, item_embedding):
    uidx = user_tensor.astype(jnp.int32)
    iidx = item_tensor.astype(jnp.int32)
    nu = user_embedding.shape[0]
    ni = item_embedding.shape[0]
    u2 = user_embedding.reshape(nu // 2, 2 * _D)
    i2 = item_embedding.reshape(ni // 2, 2 * _D)
    wu2, hi2 = _sc_gather(uidx >> 1, iidx >> 1, u2, i2)
    pu = jnp.broadcast_to((uidx & 1).astype(jnp.float32)[:, None], (_B, _D))
    pi_ = jnp.broadcast_to((iidx & 1).astype(jnp.float32)[:, None], (_B, _D))
    return _matmul(wu2, pu, hi2, pi_)


# speedup vs baseline: 6.2808x; 6.2524x over previous
import functools

import jax
import jax.numpy as jnp
from jax import lax
from jax.experimental import pallas as pl
from jax.experimental.pallas import tpu as pltpu
from jax.experimental.pallas import tpu_sc as plsc

_D = 64
_B = 4096
_NC = 2
_NS = 16
_NW = _NC * _NS
_BPW = _B // _NW
_G = _BPW // 16
_C = 4
_V = 1000000
_T0 = (_V // 128) * 128
_TW = _V - _T0

_mesh = plsc.VectorSubcoreMesh(core_axis_name="c", subcore_axis_name="s")

@functools.partial(
    pl.kernel,
    out_type=(
        jax.ShapeDtypeStruct((8, _D // 8, _B), jnp.float32),
        jax.ShapeDtypeStruct((8, _D // 8, _B), jnp.float32),
    ),
    mesh=_mesh,
    scratch_types=[
        pltpu.VMEM((_BPW,), jnp.int32),
        pltpu.VMEM((_BPW,), jnp.int32),
        pltpu.VMEM((8, _D // 8, _C * 128), jnp.float32),
        pltpu.VMEM((8, _D // 8, _C * 128), jnp.float32),
        pltpu.VMEM((8, _D // 8, _BPW), jnp.float32),
        pltpu.VMEM((8, _D // 8, _BPW), jnp.float32),
        pltpu.SemaphoreType.DMA,
        pltpu.SemaphoreType.DMA,
    ],
    compiler_params=pltpu.CompilerParams(needs_layout_passes=False),
)
def _sc_gather(user_idx_hbm, item_idx_hbm, u3, i3, ut3, it3,
               wu_out, hi_out, uidx_v, iidx_v, ubuf, ibuf,
               urows_v, irows_v, usem, isem):
    wid = lax.axis_index("s") * _NC + lax.axis_index("c")
    base = wid * _BPW
    pltpu.sync_copy(user_idx_hbm.at[pl.ds(base, _BPW)], uidx_v)
    pltpu.sync_copy(item_idx_hbm.at[pl.ds(base, _BPW)], iidx_v)

    def fetch_one(tab, tailtab, buf, sem, idx, j):
        tail = idx >= _T0
        v0 = pl.multiple_of(idx & -128, 128)

        @pl.when(jnp.logical_not(tail))
        def _():
            pltpu.async_copy(tab.at[:, :, pl.ds(v0, 128)],
                             buf.at[:, :, pl.ds(j * 128, 128)], sem)

        @pl.when(tail)
        def _():
            pltpu.async_copy(tailtab,
                             buf.at[:, :, pl.ds(j * 128, 128)], sem)

    def wait_one(tab, buf, sem, j):
        pltpu.make_async_copy(tab.at[:, :, pl.ds(0, 128)],
                              buf.at[:, :, pl.ds(j * 128, 128)], sem).wait()

    lane = lax.iota(jnp.int32, 16)
    avec = [(lane + q * 16) >> 3 for q in range(4)]
    svec = [(lane + q * 16) & 7 for q in range(4)]

    def gather_group(g, _):
        uvec = uidx_v[pl.ds(g * 16, 16)]
        ivec = iidx_v[pl.ds(g * 16, 16)]
        for c in range(16 // _C):
            for j in range(_C):
                fetch_one(u3, ut3, ubuf, usem, uvec[c * _C + j], j)
                fetch_one(i3, it3, ibuf, isem, ivec[c * _C + j], j)
            for j in range(_C):
                wait_one(u3, ubuf, usem, j)
                wait_one(i3, ibuf, isem, j)
            for j in range(_C):
                uj = uvec[c * _C + j]
                ij = ivec[c * _C + j]
                b = g * 16 + c * _C + j
                uv0 = jnp.where(uj >= _T0, _T0, uj & -128)
                iv0 = jnp.where(ij >= _T0, _T0, ij & -128)
                upos = jnp.full((16,), j * 128, jnp.int32) + (uj - uv0)
                ipos = jnp.full((16,), j * 128, jnp.int32) + (ij - iv0)
                bvec = jnp.full((16,), b, jnp.int32)
                for q in range(4):
                    uvals = plsc.load_gather(ubuf, [avec[q], svec[q], upos])
                    plsc.store_scatter(urows_v, [avec[q], svec[q], bvec], uvals)
                    ivals = plsc.load_gather(ibuf, [avec[q], svec[q], ipos])
                    plsc.store_scatter(irows_v, [avec[q], svec[q], bvec], ivals)
        return 0

    lax.fori_loop(0, _G, gather_group, 0)
    pltpu.sync_copy(urows_v, wu_out.at[:, :, pl.ds(base, _BPW)])
    pltpu.sync_copy(irows_v, hi_out.at[:, :, pl.ds(base, _BPW)])


_BN = 512


def _mm_body(wu_ref, hi_ref, out_ref):
    out_ref[...] = lax.dot_general(
        wu_ref[...], hi_ref[...],
        (((0,), (0,)), ((), ())),
        preferred_element_type=jnp.float32,
    )


_matmul = pl.pallas_call(
    _mm_body,
    grid=(_B // _BN,),
    in_specs=[
        pl.BlockSpec((_D, _B), lambda j: (0, 0)),
        pl.BlockSpec((_D, _BN), lambda j: (0, j)),
    ],
    out_specs=pl.BlockSpec((_B, _BN), lambda j: (0, j)),
    out_shape=jax.ShapeDtypeStruct((_B, _B), jnp.float32),
)


@jax.jit
def kernel(user_tensor, item_tensor, user_embedding, item_embedding):
    uidx = user_tensor.astype(jnp.int32)
    iidx = item_tensor.astype(jnp.int32)
    nu = user_embedding.shape[0]
    ni = item_embedding.shape[0]
    u3 = user_embedding.T.reshape(8, _D // 8, nu)
    i3 = item_embedding.T.reshape(8, _D // 8, ni)
    ut3 = jnp.pad(u3[:, :, _T0:], ((0, 0), (0, 0), (0, 128 - _TW)))
    it3 = jnp.pad(i3[:, :, _T0:], ((0, 0), (0, 0), (0, 128 - _TW)))
    wu3, hi3 = _sc_gather(uidx, iidx, u3, i3, ut3, it3)
    return _matmul(wu3.reshape(_D, _B), hi3.reshape(_D, _B))


# speedup vs baseline: 7.2243x; 1.1502x over previous
import functools

import jax
import jax.numpy as jnp
from jax import lax
from jax.experimental import pallas as pl
from jax.experimental.pallas import tpu as pltpu
from jax.experimental.pallas import tpu_sc as plsc

_D = 64
_B = 4096
_NC = 2
_NS = 16
_NW = _NC * _NS
_BPW = _B // _NW
_G = _BPW // 16
_C = 2
_V = 1000000
_T0 = (_V // 128) * 128
_TW = _V - _T0

_mesh = plsc.VectorSubcoreMesh(core_axis_name="c", subcore_axis_name="s")

@functools.partial(
    pl.kernel,
    out_type=(
        jax.ShapeDtypeStruct((8, _D // 8, _B), jnp.float32),
        jax.ShapeDtypeStruct((8, _D // 8, _B), jnp.float32),
    ),
    mesh=_mesh,
    scratch_types=[
        pltpu.VMEM((_BPW,), jnp.int32),
        pltpu.VMEM((_BPW,), jnp.int32),
        pltpu.VMEM((8, _D // 8, 2 * _C * 128), jnp.float32),
        pltpu.VMEM((8, _D // 8, 2 * _C * 128), jnp.float32),
        pltpu.VMEM((8, _D // 8, _BPW), jnp.float32),
        pltpu.VMEM((8, _D // 8, _BPW), jnp.float32),
        pltpu.SemaphoreType.DMA,
        pltpu.SemaphoreType.DMA,
    ],
    compiler_params=pltpu.CompilerParams(needs_layout_passes=False),
)
def _sc_gather(user_idx_hbm, item_idx_hbm, u3, i3, ut3, it3,
               wu_out, hi_out, uidx_v, iidx_v, ubuf, ibuf,
               urows_v, irows_v, usem, isem):
    wid = lax.axis_index("s") * _NC + lax.axis_index("c")
    base = wid * _BPW
    pltpu.sync_copy(user_idx_hbm.at[pl.ds(base, _BPW)], uidx_v)
    pltpu.sync_copy(item_idx_hbm.at[pl.ds(base, _BPW)], iidx_v)

    def fetch_one(tab, tailtab, buf, sem, idx, j):
        tail = idx >= _T0
        v0 = pl.multiple_of(idx & -128, 128)

        @pl.when(jnp.logical_not(tail))
        def _():
            pltpu.async_copy(tab.at[:, :, pl.ds(v0, 128)],
                             buf.at[:, :, pl.ds(j * 128, 128)], sem)

        @pl.when(tail)
        def _():
            pltpu.async_copy(tailtab,
                             buf.at[:, :, pl.ds(j * 128, 128)], sem)

    def wait_one(tab, buf, sem, j):
        pltpu.make_async_copy(tab.at[:, :, pl.ds(0, 128)],
                              buf.at[:, :, pl.ds(j * 128, 128)], sem).wait()

    lane = lax.iota(jnp.int32, 16)
    avec = [(lane + q * 16) >> 3 for q in range(4)]
    svec = [(lane + q * 16) & 7 for q in range(4)]

    def gather_group(g, _):
        uvec = uidx_v[pl.ds(g * 16, 16)]
        ivec = iidx_v[pl.ds(g * 16, 16)]
        nch = 16 // _C

        def fire(c):
            for j in range(_C):
                slot = (c % 2) * _C + j
                fetch_one(u3, ut3, ubuf, usem, uvec[c * _C + j], slot)
                fetch_one(i3, it3, ibuf, isem, ivec[c * _C + j], slot)

        def drain_extract(c):
            for j in range(_C):
                slot = (c % 2) * _C + j
                wait_one(u3, ubuf, usem, slot)
                wait_one(i3, ibuf, isem, slot)
            for j in range(_C):
                slot = (c % 2) * _C + j
                uj = uvec[c * _C + j]
                ij = ivec[c * _C + j]
                b = g * 16 + c * _C + j
                uv0 = jnp.where(uj >= _T0, _T0, uj & -128)
                iv0 = jnp.where(ij >= _T0, _T0, ij & -128)
                upos = jnp.full((16,), slot * 128, jnp.int32) + (uj - uv0)
                ipos = jnp.full((16,), slot * 128, jnp.int32) + (ij - iv0)
                bvec = jnp.full((16,), b, jnp.int32)
                for q in range(4):
                    uvals = plsc.load_gather(ubuf, [avec[q], svec[q], upos])
                    plsc.store_scatter(urows_v, [avec[q], svec[q], bvec], uvals)
                    ivals = plsc.load_gather(ibuf, [avec[q], svec[q], ipos])
                    plsc.store_scatter(irows_v, [avec[q], svec[q], bvec], ivals)

        fire(0)
        for c in range(1, nch):
            fire(c)
            drain_extract(c - 1)
        drain_extract(nch - 1)
        return 0

    lax.fori_loop(0, _G, gather_group, 0)
    pltpu.sync_copy(urows_v, wu_out.at[:, :, pl.ds(base, _BPW)])
    pltpu.sync_copy(irows_v, hi_out.at[:, :, pl.ds(base, _BPW)])


_BN = 512


def _mm_body(wu_ref, hi_ref, out_ref):
    out_ref[...] = lax.dot_general(
        wu_ref[...], hi_ref[...],
        (((0,), (0,)), ((), ())),
        preferred_element_type=jnp.float32,
    )


_matmul = pl.pallas_call(
    _mm_body,
    grid=(_B // _BN,),
    in_specs=[
        pl.BlockSpec((_D, _B), lambda j: (0, 0)),
        pl.BlockSpec((_D, _BN), lambda j: (0, j)),
    ],
    out_specs=pl.BlockSpec((_B, _BN), lambda j: (0, j)),
    out_shape=jax.ShapeDtypeStruct((_B, _B), jnp.float32),
)


@jax.jit
def kernel(user_tensor, item_tensor, user_embedding, item_embedding):
    uidx = user_tensor.astype(jnp.int32)
    iidx = item_tensor.astype(jnp.int32)
    nu = user_embedding.shape[0]
    ni = item_embedding.shape[0]
    u3 = user_embedding.T.reshape(8, _D // 8, nu)
    i3 = item_embedding.T.reshape(8, _D // 8, ni)
    ut3 = jnp.pad(u3[:, :, _T0:], ((0, 0), (0, 0), (0, 128 - _TW)))
    it3 = jnp.pad(i3[:, :, _T0:], ((0, 0), (0, 0), (0, 128 - _TW)))
    wu3, hi3 = _sc_gather(uidx, iidx, u3, i3, ut3, it3)
    return _matmul(wu3.reshape(_D, _B), hi3.reshape(_D, _B))


# speedup vs baseline: 7.5541x; 1.0457x over previous
import functools

import jax
import jax.numpy as jnp
from jax import lax
from jax.experimental import pallas as pl
from jax.experimental.pallas import tpu as pltpu
from jax.experimental.pallas import tpu_sc as plsc

_D = 64
_B = 4096
_NC = 2
_NS = 16
_NW = _NC * _NS
_BPW = _B // _NW
_G = _BPW // 16
_C = 2
_V = 1000000
_T0 = (_V // 128) * 128
_TW = _V - _T0

_mesh = plsc.VectorSubcoreMesh(core_axis_name="c", subcore_axis_name="s")

@functools.partial(
    pl.kernel,
    out_type=(
        jax.ShapeDtypeStruct((8, _D // 8, _B), jnp.float32),
        jax.ShapeDtypeStruct((8, _D // 8, _B), jnp.float32),
    ),
    mesh=_mesh,
    scratch_types=[
        pltpu.VMEM((_BPW,), jnp.int32),
        pltpu.VMEM((_BPW,), jnp.int32),
        pltpu.VMEM((8, _D // 8, 3 * _C * 128), jnp.float32),
        pltpu.VMEM((8, _D // 8, 3 * _C * 128), jnp.float32),
        pltpu.VMEM((8, _D // 8, _BPW), jnp.float32),
        pltpu.VMEM((8, _D // 8, _BPW), jnp.float32),
        pltpu.SemaphoreType.DMA,
        pltpu.SemaphoreType.DMA,
    ],
    compiler_params=pltpu.CompilerParams(needs_layout_passes=False),
)
def _sc_gather(user_idx_hbm, item_idx_hbm, u3, i3, ut3, it3,
               wu_out, hi_out, uidx_v, iidx_v, ubuf, ibuf,
               urows_v, irows_v, usem, isem):
    wid = lax.axis_index("s") * _NC + lax.axis_index("c")
    base = wid * _BPW
    pltpu.sync_copy(user_idx_hbm.at[pl.ds(base, _BPW)], uidx_v)
    pltpu.sync_copy(item_idx_hbm.at[pl.ds(base, _BPW)], iidx_v)

    def fetch_one(tab, tailtab, buf, sem, idx, j):
        tail = idx >= _T0
        v0 = pl.multiple_of(idx & -128, 128)

        @pl.when(jnp.logical_not(tail))
        def _():
            pltpu.async_copy(tab.at[:, :, pl.ds(v0, 128)],
                             buf.at[:, :, pl.ds(j * 128, 128)], sem)

        @pl.when(tail)
        def _():
            pltpu.async_copy(tailtab,
                             buf.at[:, :, pl.ds(j * 128, 128)], sem)

    def wait_one(tab, buf, sem, j):
        pltpu.make_async_copy(tab.at[:, :, pl.ds(0, 128)],
                              buf.at[:, :, pl.ds(j * 128, 128)], sem).wait()

    lane = lax.iota(jnp.int32, 16)
    avec = [(lane + q * 16) >> 3 for q in range(4)]
    svec = [(lane + q * 16) & 7 for q in range(4)]

    def gather_group(g, _):
        uvec = uidx_v[pl.ds(g * 16, 16)]
        ivec = iidx_v[pl.ds(g * 16, 16)]
        nch = 16 // _C

        def fire(c):
            for j in range(_C):
                slot = (c % 3) * _C + j
                fetch_one(u3, ut3, ubuf, usem, uvec[c * _C + j], slot)
                fetch_one(i3, it3, ibuf, isem, ivec[c * _C + j], slot)

        def drain_extract(c):
            for j in range(_C):
                slot = (c % 3) * _C + j
                wait_one(u3, ubuf, usem, slot)
                wait_one(i3, ibuf, isem, slot)
            for j in range(_C):
                slot = (c % 3) * _C + j
                uj = uvec[c * _C + j]
                ij = ivec[c * _C + j]
                b = g * 16 + c * _C + j
                uv0 = jnp.where(uj >= _T0, _T0, uj & -128)
                iv0 = jnp.where(ij >= _T0, _T0, ij & -128)
                upos = jnp.full((16,), slot * 128, jnp.int32) + (uj - uv0)
                ipos = jnp.full((16,), slot * 128, jnp.int32) + (ij - iv0)
                bvec = jnp.full((16,), b, jnp.int32)
                for q in range(4):
                    uvals = plsc.load_gather(ubuf, [avec[q], svec[q], upos])
                    plsc.store_scatter(urows_v, [avec[q], svec[q], bvec], uvals)
                    ivals = plsc.load_gather(ibuf, [avec[q], svec[q], ipos])
                    plsc.store_scatter(irows_v, [avec[q], svec[q], bvec], ivals)

        fire(0)
        fire(1)
        for c in range(2, nch):
            fire(c)
            drain_extract(c - 2)
        drain_extract(nch - 2)
        drain_extract(nch - 1)
        return 0

    lax.fori_loop(0, _G, gather_group, 0)
    pltpu.sync_copy(urows_v, wu_out.at[:, :, pl.ds(base, _BPW)])
    pltpu.sync_copy(irows_v, hi_out.at[:, :, pl.ds(base, _BPW)])


_BN = 512


def _mm_body(wu_ref, hi_ref, out_ref):
    out_ref[...] = lax.dot_general(
        wu_ref[...], hi_ref[...],
        (((0,), (0,)), ((), ())),
        preferred_element_type=jnp.float32,
    )


_matmul = pl.pallas_call(
    _mm_body,
    grid=(_B // _BN,),
    in_specs=[
        pl.BlockSpec((_D, _B), lambda j: (0, 0)),
        pl.BlockSpec((_D, _BN), lambda j: (0, j)),
    ],
    out_specs=pl.BlockSpec((_B, _BN), lambda j: (0, j)),
    out_shape=jax.ShapeDtypeStruct((_B, _B), jnp.float32),
)


@jax.jit
def kernel(user_tensor, item_tensor, user_embedding, item_embedding):
    uidx = user_tensor.astype(jnp.int32)
    iidx = item_tensor.astype(jnp.int32)
    nu = user_embedding.shape[0]
    ni = item_embedding.shape[0]
    u3 = user_embedding.T.reshape(8, _D // 8, nu)
    i3 = item_embedding.T.reshape(8, _D // 8, ni)
    ut3 = jnp.pad(u3[:, :, _T0:], ((0, 0), (0, 0), (0, 128 - _TW)))
    it3 = jnp.pad(i3[:, :, _T0:], ((0, 0), (0, 0), (0, 128 - _TW)))
    wu3, hi3 = _sc_gather(uidx, iidx, u3, i3, ut3, it3)
    return _matmul(wu3.reshape(_D, _B), hi3.reshape(_D, _B))
